# Initial kernel scaffold; baseline (speedup 1.0000x reference)
import jax
import jax.numpy as jnp
from jax.experimental import pallas as pl


def kernel(x, codebook_weight):
    raise NotImplementedError("write your pallas kernel here")


# speedup vs baseline: 4.4781x; 4.4781x over previous
import functools
import jax
import jax.numpy as jnp
from jax import lax
from jax.experimental import pallas as pl
from jax.experimental.pallas import tpu as pltpu
from jax.experimental.pallas import tpu_sc as plsc

_BETA = 0.25
_D = 32
_N = 8192
_B = 1024
_CHUNK = 512
_NSTEP = _N // _CHUNK


def _argmin_body(x_ref, cb_ref, idx_ref, loss_ref, cbn_ref,
                 z_ref, best_d_ref, best_i_ref):
    j = pl.program_id(0)

    @pl.when(j == 0)
    def _init():
        z = x_ref[...]
        zn = jnp.sqrt(jnp.sum(z * z, axis=1, keepdims=True))
        z = z / jnp.maximum(zn, 1e-12)
        z_ref[:, :_D] = z
        z_ref[:, _D:] = jnp.ones((_B, 1), jnp.float32)
        best_d_ref[...] = jnp.full((_B, 1), jnp.inf, jnp.float32)
        best_i_ref[...] = jnp.zeros((_B, 1), jnp.int32)

    cbj = cb_ref[...]
    cn = jnp.sqrt(jnp.sum(cbj * cbj, axis=1, keepdims=True))
    cbn = cbj / jnp.maximum(cn, 1e-12)
    cbn_ref[:, :_D] = cbn
    cn2 = jnp.sum(cbn * cbn, axis=1, keepdims=True)
    aug = jnp.concatenate([-2.0 * cbn, cn2], axis=1)
    d = jax.lax.dot_general(
        z_ref[...], aug, (((1,), (1,)), ((), ())),
        preferred_element_type=jnp.float32,
        precision=jax.lax.Precision.HIGHEST)
    m = jnp.min(d, axis=1, keepdims=True)
    a = jnp.argmin(d, axis=1).astype(jnp.int32)[:, None] + j * _CHUNK
    upd = m < best_d_ref[...]
    best_i_ref[...] = jnp.where(upd, a, best_i_ref[...])
    best_d_ref[...] = jnp.where(upd, m, best_d_ref[...])

    @pl.when(j == _NSTEP - 1)
    def _fin():
        idx_ref[...] = best_i_ref[...]
        zn2 = jnp.sum(z_ref[:, :_D] * z_ref[:, :_D], axis=1, keepdims=True)
        row = best_d_ref[...] + zn2
        mean = jnp.sum(row) / (_B * _D)
        loss_ref[...] = jnp.reshape(_BETA * mean + mean, (1, 1))


def _tc_argmin(xf, codebook_weight):
    return pl.pallas_call(
        _argmin_body,
        grid=(_NSTEP,),
        in_specs=[
            pl.BlockSpec((_B, _D), lambda j: (0, 0)),
            pl.BlockSpec((_CHUNK, _D), lambda j: (j, 0)),
        ],
        out_specs=[
            pl.BlockSpec((_B, 1), lambda j: (0, 0)),
            pl.BlockSpec((1, 1), lambda j: (0, 0)),
            pl.BlockSpec((_CHUNK, 128), lambda j: (j, 0)),
        ],
        out_shape=[
            jax.ShapeDtypeStruct((_B, 1), jnp.int32),
            jax.ShapeDtypeStruct((1, 1), jnp.float32),
            jax.ShapeDtypeStruct((_N, 128), jnp.float32),
        ],
        scratch_shapes=[
            pltpu.VMEM((_B, _D + 1), jnp.float32),
            pltpu.VMEM((_B, 1), jnp.float32),
            pltpu.VMEM((_B, 1), jnp.int32),
        ],
    )(xf, codebook_weight)


def _sc_gather(table, idx):
    info = plsc.get_sparse_core_info()
    nc, ns = info.num_cores, info.num_subcores
    nw = nc * ns
    bpw = _B // nw
    mesh = plsc.VectorSubcoreMesh(core_axis_name="c", subcore_axis_name="s")

    @functools.partial(
        pl.kernel, mesh=mesh,
        out_type=jax.ShapeDtypeStruct((_B, 128), jnp.float32),
        scratch_types=[
            pltpu.VMEM((bpw,), jnp.int32),
            pltpu.VMEM((bpw, 128), jnp.float32),
            pltpu.SemaphoreType.DMA,
        ],
    )
    def k(table_hbm, idx_hbm, out_hbm, idx_v, rows_v, sem):
        wid = lax.axis_index("s") * nc + lax.axis_index("c")
        base = wid * bpw
        pltpu.sync_copy(idx_hbm.at[pl.ds(base, bpw)], idx_v)
        pltpu.async_copy(table_hbm.at[idx_v], rows_v, sem).wait()
        pltpu.sync_copy(rows_v, out_hbm.at[pl.ds(base, bpw)])

    return k(table, idx)


def kernel(x, codebook_weight):
    xf = x.reshape(-1, _D)
    idx, loss, cbn = _tc_argmin(xf, codebook_weight)
    zq = _sc_gather(cbn, idx.reshape(_B))[:, :_D]
    return (zq.reshape(x.shape), loss.reshape(()),
            idx.reshape(x.shape[:-1]))


# speedup vs baseline: 6.6652x; 1.4884x over previous
import functools
import jax
import jax.numpy as jnp
from jax import lax
from jax.experimental import pallas as pl
from jax.experimental.pallas import tpu as pltpu
from jax.experimental.pallas import tpu_sc as plsc

_BETA = 0.25
_D = 32
_N = 8192
_B = 1024
_CHUNK = 512
_NSTEP = _N // _CHUNK


def _argmin_body(x_ref, cb_ref, idx_ref, loss_ref, cbn_ref,
                 z_ref, best_d_ref, best_i_ref):
    j = pl.program_id(0)

    @pl.when(j == 0)
    def _init():
        z = x_ref[...]
        zn = jnp.sqrt(jnp.sum(z * z, axis=1, keepdims=True))
        z = z / jnp.maximum(zn, 1e-12)
        z_ref[:, :_D] = z
        z_ref[:, _D:] = jnp.ones((_B, 1), jnp.float32)
        best_d_ref[...] = jnp.full((1, _B), jnp.inf, jnp.float32)
        best_i_ref[...] = jnp.zeros((1, _B), jnp.int32)

    cbj = cb_ref[...]
    cn = jnp.sqrt(jnp.sum(cbj * cbj, axis=1, keepdims=True))
    cbn = cbj / jnp.maximum(cn, 1e-12)
    cbn_ref[:, :_D] = cbn
    cn2 = jnp.sum(cbn * cbn, axis=1, keepdims=True)
    aug = jnp.concatenate([-2.0 * cbn, cn2], axis=1)
    d = jax.lax.dot_general(
        aug, z_ref[...], (((1,), (1,)), ((), ())),
        preferred_element_type=jnp.float32,
        precision=jax.lax.Precision.HIGHEST)
    m = jnp.min(d, axis=0, keepdims=True)
    a = jnp.argmin(d, axis=0).astype(jnp.int32)[None, :] + j * _CHUNK
    upd = m < best_d_ref[...]
    best_i_ref[...] = jnp.where(upd, a, best_i_ref[...])
    best_d_ref[...] = jnp.where(upd, m, best_d_ref[...])

    @pl.when(j == _NSTEP - 1)
    def _fin():
        idx_ref[...] = best_i_ref[...]
        z = z_ref[:, :_D]
        tot = jnp.sum(best_d_ref[...]) + jnp.sum(z * z)
        mean = tot / (_B * _D)
        loss_ref[...] = jnp.reshape(_BETA * mean + mean, (1, 1))


def _tc_argmin(xf, codebook_weight):
    return pl.pallas_call(
        _argmin_body,
        grid=(_NSTEP,),
        in_specs=[
            pl.BlockSpec((_B, _D), lambda j: (0, 0)),
            pl.BlockSpec((_CHUNK, _D), lambda j: (j, 0)),
        ],
        out_specs=[
            pl.BlockSpec((1, _B), lambda j: (0, 0)),
            pl.BlockSpec((1, 1), lambda j: (0, 0)),
            pl.BlockSpec((_CHUNK, 128), lambda j: (j, 0)),
        ],
        out_shape=[
            jax.ShapeDtypeStruct((1, _B), jnp.int32),
            jax.ShapeDtypeStruct((1, 1), jnp.float32),
            jax.ShapeDtypeStruct((_N, 128), jnp.float32),
        ],
        scratch_shapes=[
            pltpu.VMEM((_B, _D + 1), jnp.float32),
            pltpu.VMEM((1, _B), jnp.float32),
            pltpu.VMEM((1, _B), jnp.int32),
        ],
    )(xf, codebook_weight)


def _sc_gather(table, idx):
    info = plsc.get_sparse_core_info()
    nc, ns = info.num_cores, info.num_subcores
    nw = nc * ns
    bpw = _B // nw
    mesh = plsc.VectorSubcoreMesh(core_axis_name="c", subcore_axis_name="s")

    @functools.partial(
        pl.kernel, mesh=mesh,
        out_type=jax.ShapeDtypeStruct((_B, 128), jnp.float32),
        scratch_types=[
            pltpu.VMEM((bpw,), jnp.int32),
            pltpu.VMEM((bpw, 128), jnp.float32),
            pltpu.SemaphoreType.DMA,
        ],
    )
    def k(table_hbm, idx_hbm, out_hbm, idx_v, rows_v, sem):
        wid = lax.axis_index("s") * nc + lax.axis_index("c")
        base = wid * bpw
        pltpu.sync_copy(idx_hbm.at[pl.ds(base, bpw)], idx_v)
        pltpu.async_copy(table_hbm.at[idx_v], rows_v, sem).wait()
        pltpu.sync_copy(rows_v, out_hbm.at[pl.ds(base, bpw)])

    return k(table, idx)


def kernel(x, codebook_weight):
    xf = x.reshape(-1, _D)
    idx, loss, cbn = _tc_argmin(xf, codebook_weight)
    zq = _sc_gather(cbn, idx.reshape(_B))[:, :_D]
    return (zq.reshape(x.shape), loss.reshape(()),
            idx.reshape(x.shape[:-1]))


# speedup vs baseline: 8.0904x; 1.2138x over previous
import functools
import jax
import jax.numpy as jnp
from jax import lax
from jax.experimental import pallas as pl
from jax.experimental.pallas import tpu as pltpu
from jax.experimental.pallas import tpu_sc as plsc

_BETA = 0.25
_D = 32
_N = 8192
_B = 1024
_CHUNK = 8192
_NSTEP = _N // _CHUNK
_SUB = 256
_NSUB = _CHUNK // _SUB


_K = _D + 1


def _split3(v):
    v1 = v.astype(jnp.bfloat16)
    r = v - v1.astype(jnp.float32)
    v2 = r.astype(jnp.bfloat16)
    r2 = r - v2.astype(jnp.float32)
    v3 = r2.astype(jnp.bfloat16)
    return v1, v2, v3


def _argmin_body(x_ref, cb_ref, idx_ref, loss_ref, cbn_ref,
                 zf_ref, zk_ref, best_d_ref, best_i_ref):
    j = pl.program_id(0)

    @pl.when(j == 0)
    def _init():
        z = x_ref[...]
        zn = jnp.sqrt(jnp.sum(z * z, axis=1, keepdims=True))
        z = z / jnp.maximum(zn, 1e-12)
        zf_ref[...] = z
        zaug = jnp.concatenate([z, jnp.ones((_B, 1), jnp.float32)], axis=1)
        z1, z2, z3 = _split3(zaug)
        zk_ref[...] = jnp.concatenate([z1, z1, z1, z2, z2, z3], axis=1)
        best_d_ref[...] = jnp.full((1, _B), jnp.inf, jnp.float32)
        best_i_ref[...] = jnp.zeros((1, _B), jnp.int32)

    cbj = cb_ref[...]
    cn = jnp.sqrt(jnp.sum(cbj * cbj, axis=1, keepdims=True))
    cbn = cbj / jnp.maximum(cn, 1e-12)
    cbn_ref[:, :_D] = cbn
    cn2 = jnp.sum(cbn * cbn, axis=1, keepdims=True)
    aug = jnp.concatenate([-2.0 * cbn, cn2], axis=1)
    a1, a2, a3 = _split3(aug)
    augk = jnp.concatenate([a1, a2, a3, a1, a2, a1], axis=1)
    zk = zk_ref[...]
    for s in range(_NSUB):
        d = jax.lax.dot_general(
            augk[s * _SUB:(s + 1) * _SUB, :], zk, (((1,), (1,)), ((), ())),
            preferred_element_type=jnp.float32)
        m = jnp.min(d, axis=0, keepdims=True)
        a = (jnp.argmin(d, axis=0).astype(jnp.int32)[None, :]
             + j * _CHUNK + s * _SUB)
        upd = m < best_d_ref[...]
        best_i_ref[...] = jnp.where(upd, a, best_i_ref[...])
        best_d_ref[...] = jnp.where(upd, m, best_d_ref[...])

    @pl.when(j == _NSTEP - 1)
    def _fin():
        idx_ref[...] = best_i_ref[...]
        z = zf_ref[...]
        tot = jnp.sum(best_d_ref[...]) + jnp.sum(z * z)
        mean = tot / (_B * _D)
        loss_ref[...] = jnp.reshape(_BETA * mean + mean, (1, 1))


def _tc_argmin(xf, codebook_weight):
    return pl.pallas_call(
        _argmin_body,
        grid=(_NSTEP,),
        in_specs=[
            pl.BlockSpec((_B, _D), lambda j: (0, 0)),
            pl.BlockSpec((_CHUNK, _D), lambda j: (j, 0)),
        ],
        out_specs=[
            pl.BlockSpec((1, _B), lambda j: (0, 0)),
            pl.BlockSpec((1, 1), lambda j: (0, 0)),
            pl.BlockSpec((_CHUNK, 128), lambda j: (j, 0)),
        ],
        out_shape=[
            jax.ShapeDtypeStruct((1, _B), jnp.int32),
            jax.ShapeDtypeStruct((1, 1), jnp.float32),
            jax.ShapeDtypeStruct((_N, 128), jnp.float32),
        ],
        scratch_shapes=[
            pltpu.VMEM((_B, _D), jnp.float32),
            pltpu.VMEM((_B, 6 * _K), jnp.bfloat16),
            pltpu.VMEM((1, _B), jnp.float32),
            pltpu.VMEM((1, _B), jnp.int32),
        ],
    )(xf, codebook_weight)


def _sc_gather(table, idx):
    info = plsc.get_sparse_core_info()
    nc, ns = info.num_cores, info.num_subcores
    nw = nc * ns
    bpw = _B // nw
    mesh = plsc.VectorSubcoreMesh(core_axis_name="c", subcore_axis_name="s")

    @functools.partial(
        pl.kernel, mesh=mesh,
        out_type=jax.ShapeDtypeStruct((_B, 128), jnp.float32),
        scratch_types=[
            pltpu.VMEM((bpw,), jnp.int32),
            pltpu.VMEM((bpw, 128), jnp.float32),
            pltpu.SemaphoreType.DMA,
        ],
    )
    def k(table_hbm, idx_hbm, out_hbm, idx_v, rows_v, sem):
        wid = lax.axis_index("s") * nc + lax.axis_index("c")
        base = wid * bpw
        pltpu.sync_copy(idx_hbm.at[pl.ds(base, bpw)], idx_v)
        pltpu.async_copy(table_hbm.at[idx_v], rows_v, sem).wait()
        pltpu.sync_copy(rows_v, out_hbm.at[pl.ds(base, bpw)])

    return k(table, idx)


def kernel(x, codebook_weight):
    xf = x.reshape(-1, _D)
    idx, loss, cbn = _tc_argmin(xf, codebook_weight)
    zq = _sc_gather(cbn, idx.reshape(_B))[:, :_D]
    return (zq.reshape(x.shape), loss.reshape(()),
            idx.reshape(x.shape[:-1]))


# speedup vs baseline: 8.8067x; 1.0885x over previous
import functools
import jax
import jax.numpy as jnp
from jax import lax
from jax.experimental import pallas as pl
from jax.experimental.pallas import tpu as pltpu
from jax.experimental.pallas import tpu_sc as plsc

_BETA = 0.25
_D = 32
_N = 8192
_B = 1024
_CHUNK = 8192
_NSTEP = _N // _CHUNK
_SUB = 256
_NSUB = _CHUNK // _SUB


_K = _D + 1


def _split3(v):
    v1 = v.astype(jnp.bfloat16)
    r = v - v1.astype(jnp.float32)
    v2 = r.astype(jnp.bfloat16)
    r2 = r - v2.astype(jnp.float32)
    v3 = r2.astype(jnp.bfloat16)
    return v1, v2, v3


def _argmin_body(x_ref, cb_ref, idx_ref, loss_ref, cbn_ref,
                 zf_ref, zk_ref, best_d_ref, best_i_ref):
    j = pl.program_id(0)

    @pl.when(j == 0)
    def _init():
        z = x_ref[...]
        zn = jnp.sqrt(jnp.sum(z * z, axis=1, keepdims=True))
        z = z / jnp.maximum(zn, 1e-12)
        zf_ref[...] = z
        zaug = jnp.concatenate([z, jnp.ones((_B, 1), jnp.float32)], axis=1)
        z1, z2, z3 = _split3(zaug)
        zk_ref[...] = jnp.concatenate([z1, z1, z1, z2, z2, z3], axis=1)
        best_d_ref[...] = jnp.full((1, _B), jnp.inf, jnp.float32)
        best_i_ref[...] = jnp.zeros((1, _B), jnp.int32)

    cbj = cb_ref[...]
    s2 = jnp.sum(cbj * cbj, axis=1, keepdims=True)
    cn = jnp.maximum(jnp.sqrt(s2), 1e-12)
    cbn = cbj / cn
    cbn_ref[:, :_D] = cbn
    cn2 = s2 / (cn * cn)
    aug = jnp.concatenate([-2.0 * cbn, cn2], axis=1)
    a1, a2, a3 = _split3(aug)
    augk = jnp.concatenate([a1, a2, a3, a1, a2, a1], axis=1)
    zk = zk_ref[...]
    iota = jax.lax.broadcasted_iota(jnp.int32, (_SUB, _B), 0)
    big = jnp.int32(2 ** 30)

    def _dot(s):
        return jax.lax.dot_general(
            augk[s * _SUB:(s + 1) * _SUB, :], zk, (((1,), (1,)), ((), ())),
            preferred_element_type=jnp.float32)

    def _reduce(d, s):
        m = jnp.min(d, axis=0, keepdims=True)
        a = (jnp.min(jnp.where(d == m, iota, big), axis=0)[None, :]
             + j * _CHUNK + s * _SUB)
        upd = m < best_d_ref[...]
        best_i_ref[...] = jnp.where(upd, a, best_i_ref[...])
        best_d_ref[...] = jnp.where(upd, m, best_d_ref[...])

    d_prev = _dot(0)
    for s in range(1, _NSUB):
        d_cur = _dot(s)
        _reduce(d_prev, s - 1)
        d_prev = d_cur
    _reduce(d_prev, _NSUB - 1)

    @pl.when(j == _NSTEP - 1)
    def _fin():
        idx_ref[...] = best_i_ref[...]
        z = zf_ref[...]
        tot = jnp.sum(best_d_ref[...]) + jnp.sum(z * z)
        mean = tot / (_B * _D)
        loss_ref[...] = jnp.reshape(_BETA * mean + mean, (1, 1))


def _tc_argmin(xf, codebook_weight):
    return pl.pallas_call(
        _argmin_body,
        grid=(_NSTEP,),
        in_specs=[
            pl.BlockSpec((_B, _D), lambda j: (0, 0)),
            pl.BlockSpec((_CHUNK, _D), lambda j: (j, 0)),
        ],
        out_specs=[
            pl.BlockSpec((1, _B), lambda j: (0, 0)),
            pl.BlockSpec((1, 1), lambda j: (0, 0)),
            pl.BlockSpec((_CHUNK, 128), lambda j: (j, 0)),
        ],
        out_shape=[
            jax.ShapeDtypeStruct((1, _B), jnp.int32),
            jax.ShapeDtypeStruct((1, 1), jnp.float32),
            jax.ShapeDtypeStruct((_N, 128), jnp.float32),
        ],
        scratch_shapes=[
            pltpu.VMEM((_B, _D), jnp.float32),
            pltpu.VMEM((_B, 6 * _K), jnp.bfloat16),
            pltpu.VMEM((1, _B), jnp.float32),
            pltpu.VMEM((1, _B), jnp.int32),
        ],
    )(xf, codebook_weight)


def _sc_gather(table, idx):
    info = plsc.get_sparse_core_info()
    nc, ns = info.num_cores, info.num_subcores
    nw = nc * ns
    bpw = _B // nw
    mesh = plsc.VectorSubcoreMesh(core_axis_name="c", subcore_axis_name="s")

    @functools.partial(
        pl.kernel, mesh=mesh,
        out_type=jax.ShapeDtypeStruct((_B, 128), jnp.float32),
        scratch_types=[
            pltpu.VMEM((bpw,), jnp.int32),
            pltpu.VMEM((bpw, 128), jnp.float32),
            pltpu.SemaphoreType.DMA,
        ],
    )
    def k(table_hbm, idx_hbm, out_hbm, idx_v, rows_v, sem):
        wid = lax.axis_index("s") * nc + lax.axis_index("c")
        base = wid * bpw
        pltpu.sync_copy(idx_hbm.at[pl.ds(base, bpw)], idx_v)
        pltpu.async_copy(table_hbm.at[idx_v], rows_v, sem).wait()
        pltpu.sync_copy(rows_v, out_hbm.at[pl.ds(base, bpw)])

    return k(table, idx)


def kernel(x, codebook_weight):
    xf = x.reshape(-1, _D)
    idx, loss, cbn = _tc_argmin(xf, codebook_weight)
    zq = _sc_gather(cbn, idx.reshape(_B))[:, :_D]
    return (zq.reshape(x.shape), loss.reshape(()),
            idx.reshape(x.shape[:-1]))
